# baseline (speedup 1.0000x reference)
import functools

import jax
import jax.numpy as jnp
from jax import lax
from jax.experimental import pallas as pl
from jax.experimental.pallas import tpu as pltpu
from jax.experimental.pallas import tpu_sc as plsc

D_MODEL = 64
NUM_WORKERS = 16
CHUNK = 1600


@functools.cache
def _build(n_flat: int):
    b_per_w = n_flat // NUM_WORKERS
    n_chunks = b_per_w // CHUNK
    mesh = plsc.VectorSubcoreMesh(
        core_axis_name="c", subcore_axis_name="s", num_cores=1
    )

    @functools.partial(
        pl.kernel,
        mesh=mesh,
        compiler_params=pltpu.CompilerParams(
            use_tc_tiling_on_sc=False,
            skip_device_barrier=True,
            disable_bounds_checks=True,
            disable_semaphore_checks=True,
            has_side_effects=True,
        ),
        out_type=jax.ShapeDtypeStruct((n_flat, 128), jnp.float32),
        scratch_types=[
            pltpu.VMEM((CHUNK,), jnp.int32),
            pltpu.VMEM((CHUNK, D_MODEL), jnp.float32),
            pltpu.SemaphoreType.DMA,
        ],
    )
    def emb_kernel(idx_hbm, table_hbm, out_hbm, idx_v, rows_v, sem):
        wid = lax.axis_index("s") + lax.axis_index("c") * 0
        base = wid * b_per_w

        def body(i, carry):
            off = base + i * CHUNK
            pltpu.sync_copy(idx_hbm.at[pl.ds(off, CHUNK)], idx_v)
            pltpu.async_copy(table_hbm.at[idx_v], rows_v, sem).wait()
            pltpu.sync_copy(rows_v, out_hbm.at[pl.ds(off, CHUNK), pl.ds(0, D_MODEL)])
            return carry

        lax.fori_loop(0, n_chunks, body, 0)

    return emb_kernel


def kernel(token_ids, weight):
    b, l = token_ids.shape
    flat = token_ids.reshape(-1).astype(jnp.int32)
    out = _build(b * l)(flat, weight)
    return out[:, :D_MODEL].reshape(b, l, D_MODEL)


# speedup vs baseline: 1.1382x; 1.1382x over previous
import functools

import jax
import jax.numpy as jnp
from jax import lax
from jax.experimental import pallas as pl
from jax.experimental.pallas import tpu as pltpu
from jax.experimental.pallas import tpu_sc as plsc

D_MODEL = 64
NUM_WORKERS = 32
CHUNK = 800


@functools.cache
def _build(n_flat: int):
    b_per_w = n_flat // NUM_WORKERS
    n_pairs = b_per_w // (2 * CHUNK)
    mesh = plsc.VectorSubcoreMesh(core_axis_name="c", subcore_axis_name="s")

    @functools.partial(
        pl.kernel,
        mesh=mesh,
        compiler_params=pltpu.CompilerParams(use_tc_tiling_on_sc=False),
        out_type=jax.ShapeDtypeStruct((n_flat, 128), jnp.float32),
        scratch_types=[
            pltpu.VMEM((CHUNK,), jnp.int32),
            pltpu.VMEM((CHUNK,), jnp.int32),
            pltpu.VMEM((CHUNK, D_MODEL), jnp.float32),
            pltpu.VMEM((CHUNK, D_MODEL), jnp.float32),
            pltpu.SemaphoreType.DMA,
            pltpu.SemaphoreType.DMA,
        ],
    )
    def emb_kernel(idx_hbm, table_hbm, out_hbm, idx_a, idx_b, rows_a, rows_b,
                   sem_a, sem_b):
        wid = lax.axis_index("s") * 2 + lax.axis_index("c")
        base = wid * b_per_w

        def body(i, carry):
            off_a = base + (2 * i) * CHUNK
            off_b = off_a + CHUNK
            pltpu.sync_copy(idx_hbm.at[pl.ds(off_a, CHUNK)], idx_a)
            g_a = pltpu.async_copy(table_hbm.at[idx_a], rows_a, sem_a)
            pltpu.sync_copy(idx_hbm.at[pl.ds(off_b, CHUNK)], idx_b)
            g_b = pltpu.async_copy(table_hbm.at[idx_b], rows_b, sem_b)
            g_a.wait()
            pltpu.sync_copy(rows_a, out_hbm.at[pl.ds(off_a, CHUNK), pl.ds(0, D_MODEL)])
            g_b.wait()
            pltpu.sync_copy(rows_b, out_hbm.at[pl.ds(off_b, CHUNK), pl.ds(0, D_MODEL)])
            return carry

        lax.fori_loop(0, n_pairs, body, 0)

    return emb_kernel


def kernel(token_ids, weight):
    b, l = token_ids.shape
    flat = token_ids.reshape(-1).astype(jnp.int32)
    out = _build(b * l)(flat, weight)
    return out[:, :D_MODEL].reshape(b, l, D_MODEL)
